# Initial kernel scaffold; baseline (speedup 1.0000x reference)
import jax
import jax.numpy as jnp
from jax.experimental import pallas as pl


def kernel(z, codebook):
    raise NotImplementedError("write your pallas kernel here")


# speedup vs baseline: 1.2649x; 1.2649x over previous
import functools

import jax
import jax.numpy as jnp
from jax import lax
from jax.experimental import pallas as pl
from jax.experimental.pallas import tpu as pltpu
from jax.experimental.pallas import tpu_sc as plsc

_NUM_TOKENS = 8192
_TOKEN_DIM = 1024
_N_ROWS = 4096
_ROW_BLOCK = 512
_COMMITMENT_COST = 0.25


def _argmin_body(z_ref, cb_ref, ids_ref, loss_ref):
    i = pl.program_id(0)
    z = z_ref[...]
    cb = cb_ref[...]
    znorm = jnp.sum(z * z, axis=1, keepdims=True)
    cnorm = jnp.sum(cb * cb, axis=1)
    m = lax.dot_general(z, cb, (((1,), (1,)), ((), ())),
                        preferred_element_type=jnp.float32)
    d = (znorm - 2.0 * m) + cnorm[None, :]
    minval = jnp.min(d, axis=1, keepdims=True)
    cols = lax.broadcasted_iota(jnp.int32, d.shape, 1)
    idx = jnp.min(jnp.where(d == minval, cols, jnp.int32(_NUM_TOKENS)), axis=1)
    ids_ref[0, 0, :] = idx

    @pl.when(i == 0)
    def _init():
        loss_ref[...] = jnp.zeros((1, 1), jnp.float32)

    loss_ref[...] += jnp.sum(minval).reshape(1, 1)

    @pl.when(i == pl.num_programs(0) - 1)
    def _finish():
        loss_ref[...] = loss_ref[...] * (
            _COMMITMENT_COST / (_N_ROWS * _TOKEN_DIM))


_N_BLOCKS = _N_ROWS // _ROW_BLOCK

_argmin_call = pl.pallas_call(
    _argmin_body,
    grid=(_N_BLOCKS,),
    in_specs=[
        pl.BlockSpec((_ROW_BLOCK, _TOKEN_DIM), lambda i: (i, 0)),
        pl.BlockSpec((_NUM_TOKENS, _TOKEN_DIM), lambda i: (0, 0)),
    ],
    out_specs=[
        pl.BlockSpec((1, 1, _ROW_BLOCK), lambda i: (i, 0, 0)),
        pl.BlockSpec((1, 1), lambda i: (0, 0)),
    ],
    out_shape=[
        jax.ShapeDtypeStruct((_N_BLOCKS, 1, _ROW_BLOCK), jnp.int32),
        jax.ShapeDtypeStruct((1, 1), jnp.float32),
    ],
)


@functools.cache
def _make_sc_gather():
    info = plsc.get_sparse_core_info()
    nc, ns = info.num_cores, info.num_subcores
    nw = nc * ns
    b_per_w = _N_ROWS // nw
    ch = 64
    mesh = plsc.VectorSubcoreMesh(core_axis_name="c", subcore_axis_name="s")

    @functools.partial(
        pl.kernel,
        out_type=jax.ShapeDtypeStruct((_N_ROWS, _TOKEN_DIM), jnp.float32),
        mesh=mesh,
        scratch_types=[
            pltpu.VMEM((b_per_w,), jnp.int32),
            pltpu.VMEM((ch, _TOKEN_DIM), jnp.float32),
            pltpu.SemaphoreType.DMA,
        ],
    )
    def gather_k(idx_hbm, table_hbm, out_hbm, idx_v, rows_v, sem):
        wid = lax.axis_index("s") * nc + lax.axis_index("c")
        base = wid * b_per_w
        pltpu.sync_copy(idx_hbm.at[pl.ds(base, b_per_w)], idx_v)
        for c in range(b_per_w // ch):
            pltpu.async_copy(
                table_hbm.at[idx_v.at[pl.ds(c * ch, ch)]], rows_v, sem).wait()
            pltpu.sync_copy(rows_v, out_hbm.at[pl.ds(base + c * ch, ch)])

    return gather_k


def kernel(z, codebook):
    B, S, D = z.shape
    z_flat = z.reshape(-1, D)
    ids3, loss11 = _argmin_call(z_flat, codebook)
    ids_flat = ids3.reshape(-1)
    z_q = _make_sc_gather()(ids_flat, codebook).reshape(z.shape)
    token_ids = ids_flat.reshape(B, S)
    loss = loss11[0, 0]
    return (z_q, token_ids, loss)
